# baseline (speedup 1.0000x reference)
import math

import jax
import jax.numpy as jnp
from jax.experimental import pallas as pl
from jax.experimental.pallas import tpu as pltpu


def _phase1_body(nk, half, bk, item):
    def body(x_ref, w_ref, ts_ref, freqs_ref, embW_ref, embb_ref, wt_ref,
             bin_ref, h_ref, ss_acc):
        k = pl.program_id(1)

        def _masked():
            lim = item - k * bk
            cmask = jax.lax.broadcasted_iota(jnp.int32, (1, bk), 1) < lim
            rmask = jax.lax.broadcasted_iota(jnp.int32, (bk, 1), 0) < lim
            return (jnp.where(cmask, x_ref[...], 0.0),
                    jnp.where(rmask, w_ref[...], 0.0))

        def _plain():
            return x_ref[...], w_ref[...]

        xb, wb = jax.lax.cond(k == nk - 1, _masked, _plain)
        part = jnp.dot(xb, wb, preferred_element_type=jnp.float32)
        pss = jnp.sum(xb * xb, axis=1, keepdims=True)

        @pl.when(k == 0)
        def _():
            h_ref[...] = part
            ss_acc[...] = pss

        @pl.when(k > 0)
        def _():
            h_ref[...] = h_ref[...] + part
            ss_acc[...] = ss_acc[...] + pss

        @pl.when(k == nk - 1)
        def _():
            t = ts_ref[...].astype(jnp.float32)
            temp = t * freqs_ref[...]
            te = jnp.concatenate([jnp.cos(temp), jnp.sin(temp)], axis=-1)
            emb = jnp.dot(te, embW_ref[...],
                          preferred_element_type=jnp.float32) + embb_ref[...]
            contrib = jnp.dot(emb, wt_ref[...],
                              preferred_element_type=jnp.float32)
            norm = jnp.maximum(jnp.sqrt(ss_acc[...]), 1e-12)
            h_ref[...] = jnp.tanh(h_ref[...] / norm + contrib + bin_ref[...])

    return body


def _phase2_body(h_ref, w_ref, b_ref, o_ref):
    o_ref[...] = jnp.dot(h_ref[...], w_ref[...],
                         preferred_element_type=jnp.float32) + b_ref[...]


def kernel(x, timesteps, emb_W, emb_b, W_in, b_in, W_out, b_out):
    B, ITEM = x.shape
    HID = W_out.shape[0]
    TD = emb_W.shape[0]
    half = TD // 2

    bK = 3200
    NK = pl.cdiv(ITEM, bK)
    bN = 3200
    NN = pl.cdiv(ITEM, bN)

    ts2 = timesteps.reshape(B, 1)
    freqs = jnp.exp(-(math.log(10000.0) / half)
                    * jnp.arange(half, dtype=jnp.float32)).reshape(1, half)
    W_in_t = jax.lax.slice(W_in, (ITEM, 0), (ITEM + TD, HID))
    b_in2 = b_in.reshape(1, HID)
    emb_b2 = emb_b.reshape(1, TD)
    b_out2 = b_out.reshape(1, ITEM)

    NB = 2
    bB = B // NB
    h = pl.pallas_call(
        _phase1_body(NK, half, bK, ITEM),
        grid=(NB, NK),
        in_specs=[
            pl.BlockSpec((bB, bK), lambda b, k: (b, k)),
            pl.BlockSpec((bK, HID), lambda b, k: (k, 0)),
            pl.BlockSpec((bB, 1), lambda b, k: (b, 0)),
            pl.BlockSpec((1, half), lambda b, k: (0, 0)),
            pl.BlockSpec((TD, TD), lambda b, k: (0, 0)),
            pl.BlockSpec((1, TD), lambda b, k: (0, 0)),
            pl.BlockSpec((TD, HID), lambda b, k: (0, 0)),
            pl.BlockSpec((1, HID), lambda b, k: (0, 0)),
        ],
        out_specs=pl.BlockSpec((bB, HID), lambda b, k: (b, 0)),
        out_shape=jax.ShapeDtypeStruct((B, HID), jnp.float32),
        scratch_shapes=[pltpu.VMEM((bB, 1), jnp.float32)],
        compiler_params=pltpu.CompilerParams(
            dimension_semantics=("parallel", "arbitrary")),
    )(x, W_in, ts2, freqs, emb_W, emb_b2, W_in_t, b_in2)

    out = pl.pallas_call(
        _phase2_body,
        grid=(NN,),
        in_specs=[
            pl.BlockSpec((B, HID), lambda n: (0, 0)),
            pl.BlockSpec((HID, bN), lambda n: (0, n)),
            pl.BlockSpec((1, bN), lambda n: (0, n)),
        ],
        out_specs=pl.BlockSpec((B, bN), lambda n: (0, n)),
        out_shape=jax.ShapeDtypeStruct((B, ITEM), jnp.float32),
        compiler_params=pltpu.CompilerParams(
            dimension_semantics=("parallel",)),
    )(h, W_out, b_out2)

    return out


# speedup vs baseline: 1.0803x; 1.0803x over previous
import math

import jax
import jax.numpy as jnp
from jax.experimental import pallas as pl
from jax.experimental.pallas import tpu as pltpu


def _phase1_body(nk, half):
    def body(x_ref, w_ref, xt_ref, wt_tail_ref, ts_ref, freqs_ref, embW_ref,
             embb_ref, wemb_ref, bin_ref, h_ref, ss_acc):
        k = pl.program_id(1)
        xb = x_ref[...]
        part = jnp.dot(xb.astype(jnp.bfloat16), w_ref[...],
                       preferred_element_type=jnp.float32)
        pss = jnp.sum(xb * xb, axis=1, keepdims=True)

        @pl.when(k == 0)
        def _():
            h_ref[...] = part
            ss_acc[...] = pss

        @pl.when(k > 0)
        def _():
            h_ref[...] = h_ref[...] + part
            ss_acc[...] = ss_acc[...] + pss

        @pl.when(k == nk - 1)
        def _():
            xt = xt_ref[...]
            s = h_ref[...] + jnp.dot(xt, wt_tail_ref[...],
                                     preferred_element_type=jnp.float32)
            ss = ss_acc[...] + jnp.sum(xt * xt, axis=1, keepdims=True)
            t = ts_ref[...].astype(jnp.float32)
            temp = t * freqs_ref[...]
            te = jnp.concatenate([jnp.cos(temp), jnp.sin(temp)], axis=-1)
            emb = jnp.dot(te, embW_ref[...],
                          preferred_element_type=jnp.float32) + embb_ref[...]
            contrib = jnp.dot(emb, wemb_ref[...],
                              preferred_element_type=jnp.float32)
            norm = jnp.maximum(jnp.sqrt(ss), 1e-12)
            h_ref[...] = jnp.tanh(s / norm + contrib + bin_ref[...])

    return body


def _phase2_body(h_ref, w_ref, b_ref, o_ref):
    o_ref[...] = jnp.dot(h_ref[...].astype(jnp.bfloat16), w_ref[...],
                         preferred_element_type=jnp.float32) + b_ref[...]


def kernel(x, timesteps, emb_W, emb_b, W_in, b_in, W_out, b_out):
    B, ITEM = x.shape
    HID = W_out.shape[0]
    TD = emb_W.shape[0]
    half = TD // 2

    ALIGNED = (ITEM // 128) * 128
    TAIL = ITEM - ALIGNED
    bK = 1408
    NK = ALIGNED // bK
    bN = 3200
    NN = pl.cdiv(ITEM, bN)
    NB = 2
    bB = B // NB

    ts2 = timesteps.reshape(B, 1)
    freqs = jnp.exp(-(math.log(10000.0) / half)
                    * jnp.arange(half, dtype=jnp.float32)).reshape(1, half)
    W_main = jax.lax.slice(W_in, (0, 0), (ALIGNED, HID)).astype(jnp.bfloat16)
    W_tail = jax.lax.slice(W_in, (ALIGNED, 0), (ITEM, HID))
    x_tail = jax.lax.slice(x, (0, ALIGNED), (B, ITEM))
    W_emb = jax.lax.slice(W_in, (ITEM, 0), (ITEM + TD, HID))
    b_in2 = b_in.reshape(1, HID)
    emb_b2 = emb_b.reshape(1, TD)
    b_out2 = b_out.reshape(1, ITEM)
    W_out16 = W_out.astype(jnp.bfloat16)

    h = pl.pallas_call(
        _phase1_body(NK, half),
        grid=(NB, NK),
        in_specs=[
            pl.BlockSpec((bB, bK), lambda b, k: (b, k)),
            pl.BlockSpec((bK, HID), lambda b, k: (k, 0)),
            pl.BlockSpec((bB, TAIL), lambda b, k: (b, 0)),
            pl.BlockSpec((TAIL, HID), lambda b, k: (0, 0)),
            pl.BlockSpec((bB, 1), lambda b, k: (b, 0)),
            pl.BlockSpec((1, half), lambda b, k: (0, 0)),
            pl.BlockSpec((TD, TD), lambda b, k: (0, 0)),
            pl.BlockSpec((1, TD), lambda b, k: (0, 0)),
            pl.BlockSpec((TD, HID), lambda b, k: (0, 0)),
            pl.BlockSpec((1, HID), lambda b, k: (0, 0)),
        ],
        out_specs=pl.BlockSpec((bB, HID), lambda b, k: (b, 0)),
        out_shape=jax.ShapeDtypeStruct((B, HID), jnp.float32),
        scratch_shapes=[pltpu.VMEM((bB, 1), jnp.float32)],
        compiler_params=pltpu.CompilerParams(
            dimension_semantics=("parallel", "arbitrary")),
    )(x, W_main, x_tail, W_tail, ts2, freqs, emb_W, emb_b2, W_emb, b_in2)

    out = pl.pallas_call(
        _phase2_body,
        grid=(NN,),
        in_specs=[
            pl.BlockSpec((B, HID), lambda n: (0, 0)),
            pl.BlockSpec((HID, bN), lambda n: (0, n)),
            pl.BlockSpec((1, bN), lambda n: (0, n)),
        ],
        out_specs=pl.BlockSpec((B, bN), lambda n: (0, n)),
        out_shape=jax.ShapeDtypeStruct((B, ITEM), jnp.float32),
        compiler_params=pltpu.CompilerParams(
            dimension_semantics=("parallel",)),
    )(h, W_out16, b_out2)

    return out


# speedup vs baseline: 1.1426x; 1.0577x over previous
import math

import jax
import jax.numpy as jnp
from jax.experimental import pallas as pl
from jax.experimental.pallas import tpu as pltpu


def _phase1_body(nk, half):
    def body(x_ref, w_ref, xt_ref, wt_tail_ref, ts_ref, freqs_ref, embW_ref,
             embb_ref, wemb_ref, bin_ref, h_ref, ss_acc):
        k = pl.program_id(1)
        xb = x_ref[...]
        part = jnp.dot(xb.astype(jnp.bfloat16), w_ref[...],
                       preferred_element_type=jnp.float32)
        pss = jnp.sum(xb * xb, axis=1, keepdims=True)

        @pl.when(k == 0)
        def _():
            h_ref[...] = part
            ss_acc[...] = pss

        @pl.when(k > 0)
        def _():
            h_ref[...] = h_ref[...] + part
            ss_acc[...] = ss_acc[...] + pss

        @pl.when(k == nk - 1)
        def _():
            xt = xt_ref[...]
            s = h_ref[...] + jnp.dot(xt, wt_tail_ref[...],
                                     preferred_element_type=jnp.float32)
            ss = ss_acc[...] + jnp.sum(xt * xt, axis=1, keepdims=True)
            t = ts_ref[...].astype(jnp.float32)
            temp = t * freqs_ref[...]
            te = jnp.concatenate([jnp.cos(temp), jnp.sin(temp)], axis=-1)
            emb = jnp.dot(te, embW_ref[...],
                          preferred_element_type=jnp.float32) + embb_ref[...]
            contrib = jnp.dot(emb, wemb_ref[...],
                              preferred_element_type=jnp.float32)
            norm = jnp.maximum(jnp.sqrt(ss), 1e-12)
            h_ref[...] = jnp.tanh(s / norm + contrib + bin_ref[...])

    return body


def _phase2_body(h_ref, w_ref, b_ref, o_ref):
    o_ref[...] = jnp.dot(h_ref[...].astype(jnp.bfloat16), w_ref[...],
                         preferred_element_type=jnp.float32) + b_ref[...]


def kernel(x, timesteps, emb_W, emb_b, W_in, b_in, W_out, b_out):
    B, ITEM = x.shape
    HID = W_out.shape[0]
    TD = emb_W.shape[0]
    half = TD // 2

    ALIGNED = (ITEM // 128) * 128
    TAIL = ITEM - ALIGNED
    bK = 9088
    NK = ALIGNED // bK
    bN = 6400
    NN = pl.cdiv(ITEM, bN)
    NB = 2
    bB = B // NB

    ts2 = timesteps.reshape(B, 1)
    freqs = jnp.exp(-(math.log(10000.0) / half)
                    * jnp.arange(half, dtype=jnp.float32)).reshape(1, half)
    W_main = jax.lax.slice(W_in, (0, 0), (ALIGNED, HID)).astype(jnp.bfloat16)
    W_tail = jax.lax.slice(W_in, (ALIGNED, 0), (ITEM, HID))
    x_tail = jax.lax.slice(x, (0, ALIGNED), (B, ITEM))
    W_emb = jax.lax.slice(W_in, (ITEM, 0), (ITEM + TD, HID))
    b_in2 = b_in.reshape(1, HID)
    emb_b2 = emb_b.reshape(1, TD)
    b_out2 = b_out.reshape(1, ITEM)
    W_out16 = W_out.astype(jnp.bfloat16)

    h = pl.pallas_call(
        _phase1_body(NK, half),
        grid=(NB, NK),
        in_specs=[
            pl.BlockSpec((bB, bK), lambda b, k: (b, k)),
            pl.BlockSpec((bK, HID), lambda b, k: (k, 0)),
            pl.BlockSpec((bB, TAIL), lambda b, k: (b, 0)),
            pl.BlockSpec((TAIL, HID), lambda b, k: (0, 0)),
            pl.BlockSpec((bB, 1), lambda b, k: (b, 0)),
            pl.BlockSpec((1, half), lambda b, k: (0, 0)),
            pl.BlockSpec((TD, TD), lambda b, k: (0, 0)),
            pl.BlockSpec((1, TD), lambda b, k: (0, 0)),
            pl.BlockSpec((TD, HID), lambda b, k: (0, 0)),
            pl.BlockSpec((1, HID), lambda b, k: (0, 0)),
        ],
        out_specs=pl.BlockSpec((bB, HID), lambda b, k: (b, 0)),
        out_shape=jax.ShapeDtypeStruct((B, HID), jnp.float32),
        scratch_shapes=[pltpu.VMEM((bB, 1), jnp.float32)],
        compiler_params=pltpu.CompilerParams(
            dimension_semantics=("parallel", "arbitrary")),
    )(x, W_main, x_tail, W_tail, ts2, freqs, emb_W, emb_b2, W_emb, b_in2)

    out = pl.pallas_call(
        _phase2_body,
        grid=(NN,),
        in_specs=[
            pl.BlockSpec((B, HID), lambda n: (0, 0)),
            pl.BlockSpec((HID, bN), lambda n: (0, n)),
            pl.BlockSpec((1, bN), lambda n: (0, n)),
        ],
        out_specs=pl.BlockSpec((B, bN), lambda n: (0, n)),
        out_shape=jax.ShapeDtypeStruct((B, ITEM), jnp.float32),
        compiler_params=pltpu.CompilerParams(
            dimension_semantics=("parallel",)),
    )(h, W_out16, b_out2)

    return out
